# baseline (speedup 1.0000x reference)
import jax
import jax.numpy as jnp
from jax import lax
from jax.experimental import pallas as pl
from jax.experimental.pallas import tpu as pltpu
from jax.experimental.pallas import tpu_sc as plsc

N = 160000
D = 256
NH = 4
NSEG = 10000
EPS = 1e-5

BN = 8000
NC = 2
NS = 16
EDGES_PER_S = 10240
CHUNK = 128
NCH = EDGES_PER_S // CHUNK
NCH_HALF = NCH // NC
NP = NS * EDGES_PER_S
TBL = NSEG + 16
KB = 8
NG = NCH // KB


def _mlp_body(alpha_ref, x_ref, w1t_ref, w1c_ref, b1_ref, w2t_ref, b2_ref,
              g_ref, bb_ref, wa_ref, ba_ref, e_ref):
    a = alpha_ref[0, 0]
    x = x_ref[...]
    h = jnp.dot(x, w1t_ref[...], preferred_element_type=jnp.float32)
    h = jnp.maximum(h + b1_ref[...] + a * w1c_ref[...], 0.0)
    h = jnp.dot(h, w2t_ref[...], preferred_element_type=jnp.float32)
    h = jnp.maximum(h + b2_ref[...], 0.0)
    mu = jnp.mean(h, axis=-1, keepdims=True)
    d = h - mu
    var = jnp.mean(d * d, axis=-1, keepdims=True)
    hn = d * lax.rsqrt(var + EPS) * g_ref[...] + bb_ref[...]
    att = lax.dot_general(hn, wa_ref[...], (((1,), (1,)), ((), ())),
                          preferred_element_type=jnp.float32)
    att = att + ba_ref[...]
    s = 1.0 / (1.0 + jnp.exp(-att))
    e_ref[...] = jnp.exp(s)


def _tc_edge_scores(x, alpha, W1, b1, W2, b2, ln_g, ln_b, Wa, ba):
    w1t = W1[:, :D].T
    w1c = W1[:, D].reshape(1, -1)
    grid = (N // BN,)
    full = lambda shape: pl.BlockSpec(shape, lambda i: (0, 0))
    return pl.pallas_call(
        _mlp_body,
        grid=grid,
        in_specs=[
            pl.BlockSpec(memory_space=pltpu.SMEM),
            pl.BlockSpec((BN, D), lambda i: (i, 0)),
            full((D, D)),
            full((1, D)),
            full((1, D)),
            full((D, D)),
            full((1, D)),
            full((1, D)),
            full((1, D)),
            full((NH, D)),
            full((1, NH)),
        ],
        out_specs=pl.BlockSpec((BN, NH), lambda i: (i, 0)),
        out_shape=jax.ShapeDtypeStruct((N, NH), jnp.float32),
    )(alpha, x, w1t, w1c, b1.reshape(1, -1), W2.T, b2.reshape(1, -1),
      ln_g.reshape(1, -1), ln_b.reshape(1, -1), Wa, ba.reshape(1, -1))


def _sc_body(e_hbm, row_hbm, z_hbm, out_hbm, row_v, est, ecv, sbf, out_v,
             tbl_s, sem_ld, sem_sc, sem_g, sem_e):
    c = lax.axis_index("c")
    s = lax.axis_index("s")
    pltpu.sync_copy(row_hbm.at[s], row_v)
    base = c * NCH_HALF
    e_half = pltpu.async_copy(e_hbm.at[s, pl.ds(base, NCH_HALF)], ecv, sem_e)

    @pl.when(s == 0)
    def _():
        pltpu.sync_copy(z_hbm, tbl_s)

    plsc.subcore_barrier()

    ld = pltpu.async_copy(e_hbm.at[s, pl.ds(0, KB)], est.at[0], sem_ld)
    for g in range(NG):
        ld.wait()
        if g + 1 < NG:
            ld = pltpu.async_copy(
                e_hbm.at[s, pl.ds((g + 1) * KB, KB)], est.at[(g + 1) % 2],
                sem_ld)
        descs = []
        for b in range(KB):
            j = g * KB + b
            descs.append(pltpu.async_copy(
                est.at[g % 2, b], tbl_s.at[row_v.at[j]], sem_sc, add=True))
        for dc in descs:
            dc.wait()

    plsc.subcore_barrier()

    for g in range(NCH_HALF // KB):
        descs = []
        for b in range(KB):
            jl = g * KB + b
            descs.append(pltpu.async_copy(
                tbl_s.at[row_v.at[base + jl]], sbf.at[jl], sem_g))
        for dc in descs:
            dc.wait()
    e_half.wait()

    lane = lax.iota(jnp.int32, 16)

    def comp(t, carry):
        jl = t // 8
        k = (t % 8) * 16
        jf = jnp.full((16,), 0, jnp.int32) + jl
        lf = lane + k
        acc = jnp.zeros((16,), jnp.float32)
        for h in range(NH):
            hf = jnp.full((16,), h, jnp.int32)
            ev = plsc.load_gather(ecv, [jf, lf, hf])
            sv = plsc.load_gather(sbf, [jf, lf, hf])
            acc = acc + ev / sv
        out_v[jl, pl.ds(k, 16)] = acc * 0.25
        return carry

    lax.fori_loop(0, NCH_HALF * 8, comp, 0)
    pltpu.sync_copy(out_v, out_hbm.at[s, c])


def _sc_segment_norm(e_pad, row_pad, zeros_tbl):
    mesh = plsc.VectorSubcoreMesh(core_axis_name="c", subcore_axis_name="s")
    kern = pl.kernel(
        _sc_body,
        out_type=jax.ShapeDtypeStruct((NS, NC, NCH_HALF, CHUNK), jnp.float32),
        mesh=mesh,
        compiler_params=pltpu.CompilerParams(
            needs_layout_passes=False, use_tc_tiling_on_sc=False),
        scratch_types=[
            pltpu.VMEM((NCH, CHUNK), jnp.int32),
            pltpu.VMEM((2, KB, CHUNK, NH), jnp.float32),
            pltpu.VMEM((NCH_HALF, CHUNK, NH), jnp.float32),
            pltpu.VMEM((NCH_HALF, CHUNK, NH), jnp.float32),
            pltpu.VMEM((NCH_HALF, CHUNK), jnp.float32),
            pltpu.VMEM_SHARED((TBL, NH), jnp.float32),
            pltpu.SemaphoreType.DMA,
            pltpu.SemaphoreType.DMA,
            pltpu.SemaphoreType.DMA,
            pltpu.SemaphoreType.DMA,
        ],
    )
    return kern(e_pad, row_pad, zeros_tbl)


def kernel(x, row, alpha, W1, b1, W2, b2, ln_g, ln_b, Wa, ba):
    e = _tc_edge_scores(x, alpha, W1, b1, W2, b2, ln_g, ln_b, Wa, ba)
    pad = NP - N
    e_pad = jnp.concatenate([e, jnp.zeros((pad, NH), jnp.float32)], axis=0)
    row_pad = jnp.concatenate([row, jnp.full((pad,), NSEG, jnp.int32)])
    e_pad = e_pad.reshape(NS, NCH, CHUNK, NH)
    row_pad = row_pad.reshape(NS, NCH, CHUNK)
    zeros_tbl = jnp.zeros((TBL, NH), jnp.float32)
    out = _sc_segment_norm(e_pad, row_pad, zeros_tbl)
    return out.reshape(NP)[:N].reshape(N, 1)


# speedup vs baseline: 2.1299x; 2.1299x over previous
import jax
import jax.numpy as jnp
from jax import lax
from jax.experimental import pallas as pl
from jax.experimental.pallas import tpu as pltpu
from jax.experimental.pallas import tpu_sc as plsc

N = 160000
D = 256
NH = 4
NSEG = 10000
EPS = 1e-5

BN = 16000
NC = 2
NS = 16
NHC = NH // NC
EDGES_PER_S = 10240
CHUNK = 128
NCH = EDGES_PER_S // CHUNK
NP = NS * EDGES_PER_S
TBL = NSEG + 16
KB = 8


def _mlp_body(alpha_ref, x_ref, w1t_ref, w1c_ref, b1_ref, w2t_ref, b2_ref,
              g_ref, bb_ref, wa_ref, ba_ref, e_ref):
    a = alpha_ref[0, 0]
    x = x_ref[...]
    h = jnp.dot(x, w1t_ref[...], preferred_element_type=jnp.float32)
    h = jnp.maximum(h + b1_ref[...] + a * w1c_ref[...], 0.0)
    h = jnp.dot(h, w2t_ref[...], preferred_element_type=jnp.float32)
    h = jnp.maximum(h + b2_ref[...], 0.0)
    mu = jnp.mean(h, axis=-1, keepdims=True)
    d = h - mu
    var = jnp.mean(d * d, axis=-1, keepdims=True)
    hn = d * lax.rsqrt(var + EPS) * g_ref[...] + bb_ref[...]
    att = lax.dot_general(wa_ref[...], hn, (((1,), (1,)), ((), ())),
                          preferred_element_type=jnp.float32)
    att = att + ba_ref[...]
    s = 1.0 / (1.0 + jnp.exp(-att))
    e_ref[...] = jnp.exp(s)


def _tc_edge_scores(x, alpha, W1, b1, W2, b2, ln_g, ln_b, Wa, ba):
    w1t = W1[:, :D].T
    w1c = W1[:, D].reshape(1, -1)
    grid = (N // BN,)
    full = lambda shape: pl.BlockSpec(shape, lambda i: (0, 0))
    return pl.pallas_call(
        _mlp_body,
        grid=grid,
        in_specs=[
            pl.BlockSpec(memory_space=pltpu.SMEM),
            pl.BlockSpec((BN, D), lambda i: (i, 0)),
            full((D, D)),
            full((1, D)),
            full((1, D)),
            full((D, D)),
            full((1, D)),
            full((1, D)),
            full((1, D)),
            full((NH, D)),
            full((NH, 1)),
        ],
        out_specs=pl.BlockSpec((NH, BN), lambda i: (0, i)),
        out_shape=jax.ShapeDtypeStruct((NH, N), jnp.float32),
    )(alpha, x, w1t, w1c, b1.reshape(1, -1), W2.T, b2.reshape(1, -1),
      ln_g.reshape(1, -1), ln_b.reshape(1, -1), Wa, ba.reshape(-1, 1))


def _sc_body(e_hbm, row_hbm, z_hbm, out_hbm, row_v, e_v, out_v,
             tbl_v0, tbl_v1, tbl_s0, tbl_s1, sem):
    c = lax.axis_index("c")
    s = lax.axis_index("s")
    tbl_v = (tbl_v0, tbl_v1)
    tbl_s = (tbl_s0, tbl_s1)
    pltpu.sync_copy(row_hbm.at[s], row_v)
    for hh in range(NHC):
        pltpu.sync_copy(e_hbm.at[c * NHC + hh, s], e_v.at[hh])

    @pl.when(s == 0)
    def _():
        for hh in range(NHC):
            pltpu.sync_copy(z_hbm, tbl_s[hh])

    plsc.subcore_barrier()

    def scat(g, carry):
        j0 = g * KB
        descs = []
        for b in range(KB):
            idx = row_v.at[j0 + b]
            for hh in range(NHC):
                descs.append(pltpu.async_copy(
                    e_v.at[hh, j0 + b], tbl_s[hh].at[idx], sem, add=True))
        for dc in descs:
            dc.wait()
        return carry

    lax.fori_loop(0, NCH // KB, scat, 0)
    plsc.subcore_barrier()
    for hh in range(NHC):
        pltpu.sync_copy(tbl_s[hh], tbl_v[hh])

    def comp(t, carry):
        j = t // 8
        k = (t % 8) * 16
        r16 = row_v[j, pl.ds(k, 16)]
        acc = jnp.zeros((16,), jnp.float32)
        for hh in range(NHC):
            ev = e_v[hh, j, pl.ds(k, 16)]
            sv = plsc.load_gather(tbl_v[hh], [r16])
            acc = acc + ev / sv
        out_v[j, pl.ds(k, 16)] = acc * 0.25
        return carry

    lax.fori_loop(0, NCH * 8, comp, 0)
    pltpu.sync_copy(out_v, out_hbm.at[c, s])


def _sc_segment_norm(e_pad, row_pad, zeros_tbl):
    mesh = plsc.VectorSubcoreMesh(core_axis_name="c", subcore_axis_name="s")
    kern = pl.kernel(
        _sc_body,
        out_type=jax.ShapeDtypeStruct((NC, NS, NCH, CHUNK), jnp.float32),
        mesh=mesh,
        compiler_params=pltpu.CompilerParams(
            needs_layout_passes=False, use_tc_tiling_on_sc=False),
        scratch_types=[
            pltpu.VMEM((NCH, CHUNK), jnp.int32),
            pltpu.VMEM((NHC, NCH, CHUNK), jnp.float32),
            pltpu.VMEM((NCH, CHUNK), jnp.float32),
            pltpu.VMEM((TBL,), jnp.float32),
            pltpu.VMEM((TBL,), jnp.float32),
            pltpu.VMEM_SHARED((TBL,), jnp.float32),
            pltpu.VMEM_SHARED((TBL,), jnp.float32),
            pltpu.SemaphoreType.DMA,
        ],
    )
    return kern(e_pad, row_pad, zeros_tbl)


def kernel(x, row, alpha, W1, b1, W2, b2, ln_g, ln_b, Wa, ba):
    e = _tc_edge_scores(x, alpha, W1, b1, W2, b2, ln_g, ln_b, Wa, ba)
    pad = NP - N
    e_pad = jnp.concatenate([e, jnp.zeros((NH, pad), jnp.float32)], axis=1)
    row_pad = jnp.concatenate([row, jnp.full((pad,), NSEG, jnp.int32)])
    e_pad = e_pad.reshape(NH, NS, NCH, CHUNK)
    row_pad = row_pad.reshape(NS, NCH, CHUNK)
    zeros_tbl = jnp.zeros((TBL,), jnp.float32)
    out = _sc_segment_norm(e_pad, row_pad, zeros_tbl)
    out = out[0].reshape(NP) + out[1].reshape(NP)
    return out[:N].reshape(N, 1)


# speedup vs baseline: 2.1574x; 1.0129x over previous
import jax
import jax.numpy as jnp
from jax import lax
from jax.experimental import pallas as pl
from jax.experimental.pallas import tpu as pltpu
from jax.experimental.pallas import tpu_sc as plsc

N = 160000
D = 256
NH = 4
NSEG = 10000
EPS = 1e-5

BN = 16000
NC = 2
NS = 16
NHC = NH // NC
EDGES_PER_S = 10240
CHUNK = 128
NCH = EDGES_PER_S // CHUNK
NP = NS * EDGES_PER_S
TBL = NSEG + 16
KB = 16


def _mlp_body(alpha_ref, x_ref, w1t_ref, w1c_ref, b1_ref, w2t_ref, b2_ref,
              g_ref, bb_ref, wa_ref, ba_ref, e_ref):
    a = alpha_ref[0, 0]
    x = x_ref[...]
    h = jnp.dot(x, w1t_ref[...], preferred_element_type=jnp.float32)
    h = jnp.maximum(h + b1_ref[...] + a * w1c_ref[...], 0.0)
    h = jnp.dot(h, w2t_ref[...], preferred_element_type=jnp.float32)
    h = jnp.maximum(h + b2_ref[...], 0.0)
    mu = jnp.mean(h, axis=-1, keepdims=True)
    d = h - mu
    var = jnp.mean(d * d, axis=-1, keepdims=True)
    hn = d * lax.rsqrt(var + EPS) * g_ref[...] + bb_ref[...]
    att = lax.dot_general(wa_ref[...], hn, (((1,), (1,)), ((), ())),
                          preferred_element_type=jnp.float32)
    att = att + ba_ref[...]
    s = 1.0 / (1.0 + jnp.exp(-att))
    e_ref[...] = jnp.exp(s)


def _tc_edge_scores(x, alpha, W1, b1, W2, b2, ln_g, ln_b, Wa, ba):
    w1t = W1[:, :D].T
    w1c = W1[:, D].reshape(1, -1)
    grid = (N // BN,)
    full = lambda shape: pl.BlockSpec(shape, lambda i: (0, 0))
    return pl.pallas_call(
        _mlp_body,
        grid=grid,
        in_specs=[
            pl.BlockSpec(memory_space=pltpu.SMEM),
            pl.BlockSpec((BN, D), lambda i: (i, 0)),
            full((D, D)),
            full((1, D)),
            full((1, D)),
            full((D, D)),
            full((1, D)),
            full((1, D)),
            full((1, D)),
            full((NH, D)),
            full((NH, 1)),
        ],
        out_specs=pl.BlockSpec((NH, BN), lambda i: (0, i)),
        out_shape=jax.ShapeDtypeStruct((NH, N), jnp.float32),
    )(alpha, x, w1t, w1c, b1.reshape(1, -1), W2.T, b2.reshape(1, -1),
      ln_g.reshape(1, -1), ln_b.reshape(1, -1), Wa, ba.reshape(-1, 1))


def _sc_body(e_hbm, row_hbm, z_hbm, out_hbm, row_v, e_v, out_v,
             tbl_v0, tbl_v1, tbl_s0, tbl_s1, sem):
    c = lax.axis_index("c")
    s = lax.axis_index("s")
    tbl_v = (tbl_v0, tbl_v1)
    tbl_s = (tbl_s0, tbl_s1)
    pltpu.sync_copy(row_hbm.at[s], row_v)
    for hh in range(NHC):
        pltpu.sync_copy(e_hbm.at[c * NHC + hh, s], e_v.at[hh])

    @pl.when(s == 0)
    def _():
        for hh in range(NHC):
            pltpu.sync_copy(z_hbm, tbl_s[hh])

    plsc.subcore_barrier()

    def scat(g, carry):
        j0 = g * KB
        descs = []
        for b in range(KB):
            idx = row_v.at[j0 + b]
            for hh in range(NHC):
                descs.append(pltpu.async_copy(
                    e_v.at[hh, j0 + b], tbl_s[hh].at[idx], sem, add=True))
        for dc in descs:
            dc.wait()
        return carry

    lax.fori_loop(0, NCH // KB, scat, 0)
    plsc.subcore_barrier()
    for hh in range(NHC):
        pltpu.sync_copy(tbl_s[hh], tbl_v[hh])

    def comp(j, carry):
        for t8 in range(CHUNK // 16):
            k = t8 * 16
            r16 = row_v[j, pl.ds(k, 16)]
            acc = jnp.zeros((16,), jnp.float32)
            for hh in range(NHC):
                ev = e_v[hh, j, pl.ds(k, 16)]
                sv = plsc.load_gather(tbl_v[hh], [r16])
                acc = acc + ev / sv
            out_v[j, pl.ds(k, 16)] = acc * 0.25
        return carry

    lax.fori_loop(0, NCH, comp, 0)
    pltpu.sync_copy(out_v, out_hbm.at[c, s])


def _sc_segment_norm(e_pad, row_pad, zeros_tbl):
    mesh = plsc.VectorSubcoreMesh(core_axis_name="c", subcore_axis_name="s")
    kern = pl.kernel(
        _sc_body,
        out_type=jax.ShapeDtypeStruct((NC, NS, NCH, CHUNK), jnp.float32),
        mesh=mesh,
        compiler_params=pltpu.CompilerParams(
            needs_layout_passes=False, use_tc_tiling_on_sc=False),
        scratch_types=[
            pltpu.VMEM((NCH, CHUNK), jnp.int32),
            pltpu.VMEM((NHC, NCH, CHUNK), jnp.float32),
            pltpu.VMEM((NCH, CHUNK), jnp.float32),
            pltpu.VMEM((TBL,), jnp.float32),
            pltpu.VMEM((TBL,), jnp.float32),
            pltpu.VMEM_SHARED((TBL,), jnp.float32),
            pltpu.VMEM_SHARED((TBL,), jnp.float32),
            pltpu.SemaphoreType.DMA,
        ],
    )
    return kern(e_pad, row_pad, zeros_tbl)


def kernel(x, row, alpha, W1, b1, W2, b2, ln_g, ln_b, Wa, ba):
    e = _tc_edge_scores(x, alpha, W1, b1, W2, b2, ln_g, ln_b, Wa, ba)
    pad = NP - N
    e_pad = jnp.concatenate([e, jnp.zeros((NH, pad), jnp.float32)], axis=1)
    row_pad = jnp.concatenate([row, jnp.full((pad,), NSEG, jnp.int32)])
    e_pad = e_pad.reshape(NH, NS, NCH, CHUNK)
    row_pad = row_pad.reshape(NS, NCH, CHUNK)
    zeros_tbl = jnp.zeros((TBL,), jnp.float32)
    out = _sc_segment_norm(e_pad, row_pad, zeros_tbl)
    out = out[0].reshape(NP) + out[1].reshape(NP)
    return out[:N].reshape(N, 1)
